# initial kernel scaffold (unmeasured)
import jax
import jax.numpy as jnp
from jax import lax
from jax.experimental import pallas as pl
from jax.experimental.pallas import tpu as pltpu

N_DEV = 8
S = 1024
H = 8
DH = 128
D = H * DH
BLK = 64
SCALE = 0.08838834764831843


def kernel(x, Wq, K_ext, V_ext, Wo):
    def body(x_ref, wq_ref, k_ref, v_ref, wo_ref, out_ref,
             kv_ref, send_sems, recv_sems):
        my = lax.axis_index("i")

        kv_ref[0, 0] = k_ref[0].reshape(S, D).astype(jnp.bfloat16)
        kv_ref[0, 1] = v_ref[0].reshape(S, D).astype(jnp.bfloat16)

        for h in range(1, N_DEV):
            @pl.when((my < N_DEV - 1) & (my >= h - 1))
            def _():
                rdma = pltpu.make_async_remote_copy(
                    src_ref=kv_ref.at[h - 1],
                    dst_ref=kv_ref.at[h],
                    send_sem=send_sems.at[h],
                    recv_sem=recv_sems.at[h],
                    device_id=(my + 1,),
                    device_id_type=pl.DeviceIdType.MESH,
                )
                rdma.start()
                rdma.wait_send()

            @pl.when(my >= h)
            def _():
                recv = pltpu.make_async_remote_copy(
                    src_ref=kv_ref.at[h - 1],
                    dst_ref=kv_ref.at[h],
                    send_sem=send_sems.at[h],
                    recv_sem=recv_sems.at[h],
                    device_id=(my - 1,),
                    device_id_type=pl.DeviceIdType.MESH,
                )
                recv.wait_recv()

        q_all = jnp.dot(
            x_ref[0].astype(jnp.bfloat16), wq_ref[...].astype(jnp.bfloat16),
            preferred_element_type=jnp.float32,
        ) * SCALE
        qh = [q_all[:, h * DH:(h + 1) * DH].astype(jnp.bfloat16)
              for h in range(H)]

        row_blk = lax.broadcasted_iota(jnp.int32, (S, S), 0) // BLK
        col_blk = lax.broadcasted_iota(jnp.int32, (S, S), 1) // BLK
        diag_disallow = col_blk > row_blk
        NEG = jnp.float32(-1e9)

        def chunk_step(r, carry):
            ms, ls, accs = carry
            kc = kv_ref[r, 0]
            vc = kv_ref[r, 1]
            is_diag = r == 0
            new_ms, new_ls, new_accs = [], [], []
            for h in range(H):
                kh = kc[:, h * DH:(h + 1) * DH]
                s = lax.dot_general(
                    qh[h], kh, (((1,), (1,)), ((), ())),
                    preferred_element_type=jnp.float32,
                )
                s = jnp.where(jnp.logical_and(is_diag, diag_disallow), NEG, s)
                m_new = jnp.maximum(ms[h], jnp.max(s, axis=1, keepdims=True))
                p = jnp.exp(s - m_new)
                alpha = jnp.exp(ms[h] - m_new)
                vh = vc[:, h * DH:(h + 1) * DH]
                pv = lax.dot_general(
                    p.astype(jnp.bfloat16), vh, (((1,), (0,)), ((), ())),
                    preferred_element_type=jnp.float32,
                )
                new_ms.append(m_new)
                new_ls.append(ls[h] * alpha + jnp.sum(p, axis=1, keepdims=True))
                new_accs.append(accs[h] * alpha + pv)
            return tuple(new_ms), tuple(new_ls), tuple(new_accs)

        init = (
            tuple(jnp.full((S, 1), -1e30, jnp.float32) for _ in range(H)),
            tuple(jnp.zeros((S, 1), jnp.float32) for _ in range(H)),
            tuple(jnp.zeros((S, DH), jnp.float32) for _ in range(H)),
        )
        ms, ls, accs = lax.fori_loop(0, my + 1, chunk_step, init)

        ctx = jnp.concatenate(
            [accs[h] / ls[h] for h in range(H)], axis=1)
        out_ref[0] = jnp.dot(
            ctx.astype(jnp.bfloat16), wo_ref[...].astype(jnp.bfloat16),
            preferred_element_type=jnp.float32,
        )

    return pl.pallas_call(
        body,
        out_shape=jax.ShapeDtypeStruct((1, S, D), jnp.float32),
        in_specs=[pl.BlockSpec(memory_space=pltpu.VMEM)] * 5,
        out_specs=pl.BlockSpec(memory_space=pltpu.VMEM),
        scratch_shapes=[
            pltpu.VMEM((N_DEV, 2, S, D), jnp.bfloat16),
            pltpu.SemaphoreType.DMA((N_DEV,)),
            pltpu.SemaphoreType.DMA((N_DEV,)),
        ],
        compiler_params=pltpu.CompilerParams(collective_id=0),
    )(x, Wq, K_ext, V_ext, Wo)


# baseline (device time: 302025 ns/iter reference)
import jax
import jax.numpy as jnp
from jax import lax
from jax.experimental import pallas as pl
from jax.experimental.pallas import tpu as pltpu

N_DEV = 8
S = 1024
H = 8
DH = 128
BLK = 64
SCALE = 0.08838834764831843


def _attn_body(q_ref, k_ref, v_ref, mask_ref, out_ref, kv_hbm,
               stage_ref, m_ref, l_ref, acc_ref,
               send_sems, recv_sems, copy_sem):
    my = lax.axis_index("i")

    for h in range(1, N_DEV):
        @pl.when((my < N_DEV - 1) & (my >= h - 1))
        def _():
            if h == 1:
                for src, half in [(k_ref, 0), (v_ref, 1)]:
                    rdma = pltpu.make_async_remote_copy(
                        src_ref=src,
                        dst_ref=kv_hbm.at[0, half],
                        send_sem=send_sems.at[1, half],
                        recv_sem=recv_sems.at[1, half],
                        device_id=(my + 1,),
                        device_id_type=pl.DeviceIdType.MESH,
                    )
                    rdma.start()
                    rdma.wait_send()
            else:
                rdma = pltpu.make_async_remote_copy(
                    src_ref=kv_hbm.at[h - 2],
                    dst_ref=kv_hbm.at[h - 1],
                    send_sem=send_sems.at[h, 0],
                    recv_sem=recv_sems.at[h, 0],
                    device_id=(my + 1,),
                    device_id_type=pl.DeviceIdType.MESH,
                )
                rdma.start()
                rdma.wait_send()

        @pl.when(my >= h)
        def _():
            if h == 1:
                for half in [0, 1]:
                    recv = pltpu.make_async_remote_copy(
                        src_ref=kv_hbm.at[0, half],
                        dst_ref=kv_hbm.at[0, half],
                        send_sem=send_sems.at[1, half],
                        recv_sem=recv_sems.at[1, half],
                        device_id=(my - 1,),
                        device_id_type=pl.DeviceIdType.MESH,
                    )
                    recv.wait_recv()
            else:
                recv = pltpu.make_async_remote_copy(
                    src_ref=kv_hbm.at[h - 2],
                    dst_ref=kv_hbm.at[h - 1],
                    send_sem=send_sems.at[h, 0],
                    recv_sem=recv_sems.at[h, 0],
                    device_id=(my - 1,),
                    device_id_type=pl.DeviceIdType.MESH,
                )
                recv.wait_recv()

    def flash_update(h, kh, vh, s_bias):
        qh = q_ref[h]
        s = lax.dot_general(
            qh, kh, (((1,), (1,)), ((), ())),
            preferred_element_type=jnp.float32,
        )
        if s_bias is not None:
            s = s + s_bias
        m = m_ref[h]
        m_new = jnp.maximum(m, jnp.max(s, axis=1, keepdims=True))
        p = jnp.exp(s - m_new)
        alpha = jnp.exp(m - m_new)
        pv = lax.dot_general(
            p.astype(jnp.bfloat16), vh, (((1,), (0,)), ((), ())),
            preferred_element_type=jnp.float32,
        )
        m_ref[h] = m_new
        l_ref[h] = l_ref[h] * alpha + jnp.sum(p, axis=1, keepdims=True)
        acc_ref[h] = acc_ref[h] * alpha + pv

    def diag_head(h, _):
        m_ref[h] = jnp.full((S, 1), -1e30, jnp.float32)
        l_ref[h] = jnp.zeros((S, 1), jnp.float32)
        acc_ref[h] = jnp.zeros((S, DH), jnp.float32)
        flash_update(h, k_ref[h], v_ref[h], mask_ref[...].astype(jnp.float32))
        return 0

    lax.fori_loop(0, H, diag_head, 0)

    def chunk_step(r, _):
        cp = pltpu.make_async_copy(kv_hbm.at[r - 1], stage_ref, copy_sem)
        cp.start()
        cp.wait()

        def head_step(h, _):
            flash_update(h, stage_ref[0, h], stage_ref[1, h], None)
            return 0

        lax.fori_loop(0, H, head_step, 0)
        return 0

    lax.fori_loop(1, my + 1, chunk_step, 0)

    def finish(h, _):
        out_ref[h] = (acc_ref[h] / l_ref[h]).astype(jnp.bfloat16)
        return 0

    lax.fori_loop(0, H, finish, 0)


def _attn(q3, k3, v3, mask_add):
    ctx, _ = pl.pallas_call(
        _attn_body,
        out_shape=[
            jax.ShapeDtypeStruct((H, S, DH), jnp.bfloat16),
            jax.ShapeDtypeStruct((N_DEV - 1, 2, H, S, DH), jnp.bfloat16),
        ],
        in_specs=[pl.BlockSpec(memory_space=pltpu.VMEM)] * 4,
        out_specs=[
            pl.BlockSpec(memory_space=pltpu.VMEM),
            pl.BlockSpec(memory_space=pltpu.MemorySpace.HBM),
        ],
        scratch_shapes=[
            pltpu.VMEM((2, H, S, DH), jnp.bfloat16),
            pltpu.VMEM((H, S, 1), jnp.float32),
            pltpu.VMEM((H, S, 1), jnp.float32),
            pltpu.VMEM((H, S, DH), jnp.float32),
            pltpu.SemaphoreType.DMA((N_DEV, 2)),
            pltpu.SemaphoreType.DMA((N_DEV, 2)),
            pltpu.SemaphoreType.DMA,
        ],
    )(q3, k3, v3, mask_add)
    return ctx


def kernel(x, Wq, K_ext, V_ext, Wo):
    bf = jnp.bfloat16
    q = (x[0].astype(bf) @ Wq.astype(bf)).astype(jnp.float32) * SCALE
    q3 = q.reshape(S, H, DH).transpose(1, 0, 2).astype(bf)
    k3 = K_ext[0].transpose(1, 0, 2).astype(bf)
    v3 = V_ext[0].transpose(1, 0, 2).astype(bf)
    row_blk = jnp.arange(S)[:, None] // BLK
    col_blk = jnp.arange(S)[None, :] // BLK
    mask_add = jnp.where(col_blk > row_blk, -1e9, 0.0).astype(bf)
    ctx = _attn(q3, k3, v3, mask_add)
    ctx2 = ctx.transpose(1, 0, 2).reshape(S, H * DH)
    out = ctx2 @ Wo.astype(bf)
    return out.astype(jnp.float32)[None]


# device time: 211358 ns/iter; 1.4290x vs baseline; 1.4290x over previous
import jax
import jax.numpy as jnp
from jax import lax
from jax.experimental import pallas as pl
from jax.experimental.pallas import tpu as pltpu

N_DEV = 8
S = 1024
H = 8
DH = 128
BLK = 64
SCALE = 0.08838834764831843


def _attn_body(q_ref, k_ref, v_ref, mask_ref, out_ref, kv_hbm,
               stage_ref, m_ref, l_ref, acc_ref,
               send_sems, recv_sems, copy_sem):
    my = lax.axis_index("i")

    def hop1_rdma(half, target):
        src = k_ref if half == 0 else v_ref
        return pltpu.make_async_remote_copy(
            src_ref=src,
            dst_ref=kv_hbm.at[0, half],
            send_sem=send_sems.at[1, half],
            recv_sem=recv_sems.at[1, half],
            device_id=(target,),
            device_id_type=pl.DeviceIdType.MESH,
        )

    def fwd_rdma(h, target):
        return pltpu.make_async_remote_copy(
            src_ref=kv_hbm.at[h - 2],
            dst_ref=kv_hbm.at[h - 1],
            send_sem=send_sems.at[h, 0],
            recv_sem=recv_sems.at[h, 0],
            device_id=(target,),
            device_id_type=pl.DeviceIdType.MESH,
        )

    def flash_update(h, kh, vh, s_bias):
        qh = q_ref[h]
        s = lax.dot_general(
            qh, kh, (((1,), (1,)), ((), ())),
            preferred_element_type=jnp.float32,
        )
        if s_bias is not None:
            s = s + s_bias
        m = m_ref[h]
        m_new = jnp.maximum(m, jnp.max(s, axis=1, keepdims=True))
        p = jnp.exp(s - m_new)
        alpha = jnp.exp(m - m_new)
        pv = lax.dot_general(
            p.astype(jnp.bfloat16), vh, (((1,), (0,)), ((), ())),
            preferred_element_type=jnp.float32,
        )
        m_ref[h] = m_new
        l_ref[h] = l_ref[h] * alpha + jnp.sum(p, axis=1, keepdims=True)
        acc_ref[h] = acc_ref[h] * alpha + pv

    @pl.when(my < N_DEV - 1)
    def _():
        hop1_rdma(0, my + 1).start()
        hop1_rdma(1, my + 1).start()

    def diag_head(h, _):
        m_ref[h] = jnp.full((S, 1), -1e30, jnp.float32)
        l_ref[h] = jnp.zeros((S, 1), jnp.float32)
        acc_ref[h] = jnp.zeros((S, DH), jnp.float32)
        flash_update(h, k_ref[h], v_ref[h], mask_ref[...].astype(jnp.float32))
        return 0

    lax.fori_loop(0, H, diag_head, 0)

    for h in range(1, N_DEV):
        @pl.when(my >= h)
        def _():
            if h == 1:
                hop1_rdma(0, my).wait_recv()
                hop1_rdma(1, my).wait_recv()
            else:
                fwd_rdma(h, my - 1).wait_recv()

        if h + 1 < N_DEV:
            @pl.when((my < N_DEV - 1) & (my >= h))
            def _():
                fwd_rdma(h + 1, my + 1).start()

        @pl.when(my >= h)
        def _():
            cp = pltpu.make_async_copy(kv_hbm.at[h - 1], stage_ref, copy_sem)
            cp.start()
            cp.wait()

            def head_step(hh, _):
                flash_update(hh, stage_ref[0, hh], stage_ref[1, hh], None)
                return 0

            lax.fori_loop(0, H, head_step, 0)

    @pl.when(my < N_DEV - 1)
    def _():
        hop1_rdma(0, my + 1).wait_send()
        hop1_rdma(1, my + 1).wait_send()
    for h in range(2, N_DEV):
        @pl.when((my < N_DEV - 1) & (my >= h - 1))
        def _():
            fwd_rdma(h, my + 1).wait_send()

    def finish(h, _):
        out_ref[h] = (acc_ref[h] / l_ref[h]).astype(jnp.bfloat16)
        return 0

    lax.fori_loop(0, H, finish, 0)


def _attn(q3, k3, v3, mask_add):
    ctx, _ = pl.pallas_call(
        _attn_body,
        out_shape=[
            jax.ShapeDtypeStruct((H, S, DH), jnp.bfloat16),
            jax.ShapeDtypeStruct((N_DEV - 1, 2, H, S, DH), jnp.bfloat16),
        ],
        in_specs=[pl.BlockSpec(memory_space=pltpu.VMEM)] * 4,
        out_specs=[
            pl.BlockSpec(memory_space=pltpu.VMEM),
            pl.BlockSpec(memory_space=pltpu.MemorySpace.HBM),
        ],
        scratch_shapes=[
            pltpu.VMEM((2, H, S, DH), jnp.bfloat16),
            pltpu.VMEM((H, S, 1), jnp.float32),
            pltpu.VMEM((H, S, 1), jnp.float32),
            pltpu.VMEM((H, S, DH), jnp.float32),
            pltpu.SemaphoreType.DMA((N_DEV, 2)),
            pltpu.SemaphoreType.DMA((N_DEV, 2)),
            pltpu.SemaphoreType.DMA,
        ],
    )(q3, k3, v3, mask_add)
    return ctx


def kernel(x, Wq, K_ext, V_ext, Wo):
    bf = jnp.bfloat16
    q = (x[0].astype(bf) @ Wq.astype(bf)).astype(jnp.float32) * SCALE
    q3 = q.reshape(S, H, DH).transpose(1, 0, 2).astype(bf)
    k3 = K_ext[0].transpose(1, 0, 2).astype(bf)
    v3 = V_ext[0].transpose(1, 0, 2).astype(bf)
    row_blk = jnp.arange(S)[:, None] // BLK
    col_blk = jnp.arange(S)[None, :] // BLK
    mask_add = jnp.where(col_blk > row_blk, -1e9, 0.0).astype(bf)
    ctx = _attn(q3, k3, v3, mask_add)
    ctx2 = ctx.transpose(1, 0, 2).reshape(S, H * DH)
    out = ctx2 @ Wo.astype(bf)
    return out.astype(jnp.float32)[None]
